# baseline (speedup 1.0000x reference)
import jax
import jax.numpy as jnp
from jax import lax
from jax.experimental import pallas as pl
from jax.experimental.pallas import tpu as pltpu
from jax.experimental.pallas import tpu_sc as plsc

NENTITY = 1000000
NRELATION = 1000000
HIDDEN = 64
BATCH = 16384

NC = 2
NS = 16
NW = NC * NS
BPW = BATCH // NW
Q = 64
NQ = BPW // Q
DEPTH = 4

BN = 16384
NB = -(-NENTITY // BN)
NLINES = NB * BN // 2
BSH = BN.bit_length() - 1
HMASK = BN // 2 - 1


def _body(h_idx_hbm, r_idx_hbm, t_idx_hbm, gamma_hbm, ez_hbm, rz_hbm, out_hbm,
          idx_h, idx_r, idx_t, row_h, row_r, row_t, par_h, par_r, par_t,
          mh0, mr0, mt0, mh1, mr1, mt1, mh2, mr2, mt2, mh3, mr3, mt3,
          gamma_v, out_v, sem0, sem1, sem2, sem3):
    wid = lax.axis_index("s") * NC + lax.axis_index("c")
    base = wid * BPW

    pltpu.sync_copy(h_idx_hbm.at[wid], idx_h)
    pltpu.sync_copy(r_idx_hbm.at[wid], idx_r)
    pltpu.sync_copy(t_idx_hbm.at[wid], idx_t)
    pltpu.sync_copy(gamma_hbm, gamma_v)

    def prep(v, carry):
        d = pl.ds(v * 16, 16)
        for idxr, rowr, parr in ((idx_h, row_h, par_h),
                                 (idx_r, row_r, par_r),
                                 (idx_t, row_t, par_t)):
            w = idxr[d]
            rowr[d] = (lax.shift_right_logical(w, BSH) * (BN // 2)
                       + lax.bitwise_and(w, HMASK))
            parr[d] = lax.bitwise_and(lax.shift_right_logical(w, BSH - 1), 1)
        return carry

    lax.fori_loop(0, BPW // 16, prep, 0)

    bufs = ((mh0, mr0, mt0, sem0), (mh1, mr1, mt1, sem1),
            (mh2, mr2, mt2, sem2), (mh3, mr3, mt3, sem3))

    def fire(q, b):
        mh, mr, mt, sem = bufs[b]
        d = pl.ds(q * Q, Q)
        pltpu.async_copy(ez_hbm.at[row_h.at[d]], mh, sem)
        pltpu.async_copy(rz_hbm.at[row_r.at[d]], mr, sem)
        pltpu.async_copy(ez_hbm.at[row_t.at[d]], mt, sem)

    def drain(b):
        src = ez_hbm.at[pl.ds(0, Q)]
        mh, mr, mt, sem = bufs[b]
        for m in (mh, mr, mt):
            pltpu.make_async_copy(src, m, sem).wait()

    lane = lax.iota(jnp.int32, 16)
    gamma_vec = gamma_v[...]

    def compute(q, b):
        mh, mr, mt, _ = bufs[b]

        def group(g, carry):
            grow = q * Q + g * 16
            row16 = g * 16 + lane
            ch = par_h[pl.ds(grow, 16)] * 64
            cr = par_r[pl.ds(grow, 16)] * 64
            ct = par_t[pl.ds(grow, 16)] * 64
            acc = None
            for j in range(HIDDEN):
                v = jnp.abs(plsc.load_gather(mh, [row16, ch + j])
                            + plsc.load_gather(mr, [row16, cr + j])
                            - plsc.load_gather(mt, [row16, ct + j]))
                acc = v if acc is None else acc + v
            out_v[pl.ds(grow, 16)] = gamma_vec - acc
            return carry

        lax.fori_loop(0, Q // 16, group, 0)

    for q in range(DEPTH - 1):
        fire(q, q)
    for q in range(NQ):
        nxt = q + DEPTH - 1
        if nxt < NQ:
            fire(nxt, nxt % DEPTH)
        drain(q % DEPTH)
        compute(q, q % DEPTH)

    pltpu.sync_copy(out_v, out_hbm.at[pl.ds(base, BPW)])


@jax.jit
def _transe_scores(h_idx, r_idx, t_idx, gamma16, ez, rz):
    mesh = plsc.VectorSubcoreMesh(core_axis_name="c", subcore_axis_name="s")
    f = pl.kernel(
        _body, mesh=mesh,
        compiler_params=pltpu.CompilerParams(
            needs_layout_passes=False, use_tc_tiling_on_sc=True),
        out_type=jax.ShapeDtypeStruct((BATCH,), jnp.float32),
        scratch_types=[
            pltpu.VMEM((BPW,), jnp.int32),
            pltpu.VMEM((BPW,), jnp.int32),
            pltpu.VMEM((BPW,), jnp.int32),
            pltpu.VMEM((BPW,), jnp.int32),
            pltpu.VMEM((BPW,), jnp.int32),
            pltpu.VMEM((BPW,), jnp.int32),
            pltpu.VMEM((BPW,), jnp.int32),
            pltpu.VMEM((BPW,), jnp.int32),
            pltpu.VMEM((BPW,), jnp.int32),
            pltpu.VMEM((Q, 128), jnp.float32),
            pltpu.VMEM((Q, 128), jnp.float32),
            pltpu.VMEM((Q, 128), jnp.float32),
            pltpu.VMEM((Q, 128), jnp.float32),
            pltpu.VMEM((Q, 128), jnp.float32),
            pltpu.VMEM((Q, 128), jnp.float32),
            pltpu.VMEM((Q, 128), jnp.float32),
            pltpu.VMEM((Q, 128), jnp.float32),
            pltpu.VMEM((Q, 128), jnp.float32),
            pltpu.VMEM((Q, 128), jnp.float32),
            pltpu.VMEM((Q, 128), jnp.float32),
            pltpu.VMEM((Q, 128), jnp.float32),
            pltpu.VMEM((16,), jnp.float32),
            pltpu.VMEM((BPW,), jnp.float32),
            pltpu.SemaphoreType.DMA,
            pltpu.SemaphoreType.DMA,
            pltpu.SemaphoreType.DMA,
            pltpu.SemaphoreType.DMA,
        ],
    )
    return f(h_idx, r_idx, t_idx, gamma16, ez, rz)


def _tbody(x_ref, y_ref, o_ref, p_ref):
    t = x_ref[...].T
    o_ref[:, 0:HIDDEN] = t[0:BN // 2]
    o_ref[:, HIDDEN:2 * HIDDEN] = t[BN // 2:BN]
    u = y_ref[...].T
    p_ref[:, 0:HIDDEN] = u[0:BN // 2]
    p_ref[:, HIDDEN:2 * HIDDEN] = u[BN // 2:BN]


@jax.jit
def _relayout(te, tr):
    line = jax.ShapeDtypeStruct((NLINES, 128), jnp.float32)
    return pl.pallas_call(
        _tbody,
        grid=(NB,),
        in_specs=[pl.BlockSpec((HIDDEN, BN), lambda i: (0, i)),
                  pl.BlockSpec((HIDDEN, BN), lambda i: (0, i))],
        out_specs=[pl.BlockSpec((BN // 2, 128), lambda i: (i, 0)),
                   pl.BlockSpec((BN // 2, 128), lambda i: (i, 0))],
        out_shape=[line, line],
    )(te, tr)


def kernel(sample, entity_embedding, relation_embedding, gamma):
    sample = sample.astype(jnp.int32)
    h_idx = sample[:, 0].reshape(NW, BPW)
    r_idx = sample[:, 1].reshape(NW, BPW)
    t_idx = sample[:, 2].reshape(NW, BPW)
    gamma16 = jnp.broadcast_to(gamma.astype(jnp.float32), (16,))
    ez, rz = _relayout(entity_embedding.T, relation_embedding.T)
    scores = _transe_scores(h_idx, r_idx, t_idx, gamma16, ez, rz)
    return scores.reshape(BATCH, 1)


# speedup vs baseline: 1.0082x; 1.0082x over previous
import jax
import jax.numpy as jnp
from jax import lax
from jax.experimental import pallas as pl
from jax.experimental.pallas import tpu as pltpu
from jax.experimental.pallas import tpu_sc as plsc

NENTITY = 1000000
NRELATION = 1000000
HIDDEN = 64
BATCH = 16384

NC = 2
NS = 16
NW = NC * NS
BPW = BATCH // NW
Q = 128
NQ = BPW // Q

BN = 16384
NB = -(-NENTITY // BN)
NLINES = NB * BN // 2
BSH = BN.bit_length() - 1
HMASK = BN // 2 - 1


def _body(h_idx_hbm, r_idx_hbm, t_idx_hbm, gamma_hbm, ez_hbm, rz_hbm, out_hbm,
          idx_h, idx_r, idx_t, row_h, row_r, row_t, par_h, par_r, par_t,
          mh0, mr0, mt0, mh1, mr1, mt1, gamma_v, out_v, sem0, sem1):
    wid = lax.axis_index("s") * NC + lax.axis_index("c")
    base = wid * BPW

    pltpu.sync_copy(h_idx_hbm.at[wid], idx_h)
    pltpu.sync_copy(r_idx_hbm.at[wid], idx_r)
    pltpu.sync_copy(t_idx_hbm.at[wid], idx_t)
    pltpu.sync_copy(gamma_hbm, gamma_v)

    def prep(v, carry):
        d = pl.ds(v * 16, 16)
        for idxr, rowr, parr in ((idx_h, row_h, par_h),
                                 (idx_r, row_r, par_r),
                                 (idx_t, row_t, par_t)):
            w = idxr[d]
            rowr[d] = (lax.shift_right_logical(w, BSH) * (BN // 2)
                       + lax.bitwise_and(w, HMASK))
            parr[d] = lax.bitwise_and(lax.shift_right_logical(w, BSH - 1), 1)
        return carry

    lax.fori_loop(0, BPW // 16, prep, 0)

    bufs = ((mh0, mr0, mt0, sem0), (mh1, mr1, mt1, sem1))

    def fire(q, b):
        mh, mr, mt, sem = bufs[b]
        d = pl.ds(q * Q, Q)
        pltpu.async_copy(ez_hbm.at[row_h.at[d]], mh, sem)
        pltpu.async_copy(rz_hbm.at[row_r.at[d]], mr, sem)
        pltpu.async_copy(ez_hbm.at[row_t.at[d]], mt, sem)

    def drain(b):
        src = ez_hbm.at[pl.ds(0, Q)]
        mh, mr, mt, sem = bufs[b]
        for m in (mh, mr, mt):
            pltpu.make_async_copy(src, m, sem).wait()

    lane = lax.iota(jnp.int32, 16)
    gamma_vec = gamma_v[...]

    def compute(q, b):
        mh, mr, mt, _ = bufs[b]

        def group(g, carry):
            grow = q * Q + g * 16
            row16 = g * 16 + lane
            ch = par_h[pl.ds(grow, 16)] * 64
            cr = par_r[pl.ds(grow, 16)] * 64
            ct = par_t[pl.ds(grow, 16)] * 64
            acc = None
            for j in range(HIDDEN):
                v = jnp.abs(plsc.load_gather(mh, [row16, ch + j])
                            + plsc.load_gather(mr, [row16, cr + j])
                            - plsc.load_gather(mt, [row16, ct + j]))
                acc = v if acc is None else acc + v
            out_v[pl.ds(grow, 16)] = gamma_vec - acc
            return carry

        lax.fori_loop(0, Q // 16, group, 0)

    fire(0, 0)
    for q in range(NQ):
        if q + 1 < NQ:
            fire(q + 1, (q + 1) % 2)
        drain(q % 2)
        compute(q, q % 2)

    pltpu.sync_copy(out_v, out_hbm.at[pl.ds(base, BPW)])


@jax.jit
def _transe_scores(h_idx, r_idx, t_idx, gamma16, ez, rz):
    mesh = plsc.VectorSubcoreMesh(core_axis_name="c", subcore_axis_name="s")
    f = pl.kernel(
        _body, mesh=mesh,
        compiler_params=pltpu.CompilerParams(
            needs_layout_passes=False, use_tc_tiling_on_sc=True),
        out_type=jax.ShapeDtypeStruct((BATCH,), jnp.float32),
        scratch_types=[
            pltpu.VMEM((BPW,), jnp.int32),
            pltpu.VMEM((BPW,), jnp.int32),
            pltpu.VMEM((BPW,), jnp.int32),
            pltpu.VMEM((BPW,), jnp.int32),
            pltpu.VMEM((BPW,), jnp.int32),
            pltpu.VMEM((BPW,), jnp.int32),
            pltpu.VMEM((BPW,), jnp.int32),
            pltpu.VMEM((BPW,), jnp.int32),
            pltpu.VMEM((BPW,), jnp.int32),
            pltpu.VMEM((Q, 128), jnp.float32),
            pltpu.VMEM((Q, 128), jnp.float32),
            pltpu.VMEM((Q, 128), jnp.float32),
            pltpu.VMEM((Q, 128), jnp.float32),
            pltpu.VMEM((Q, 128), jnp.float32),
            pltpu.VMEM((Q, 128), jnp.float32),
            pltpu.VMEM((16,), jnp.float32),
            pltpu.VMEM((BPW,), jnp.float32),
            pltpu.SemaphoreType.DMA,
            pltpu.SemaphoreType.DMA,
        ],
    )
    return f(h_idx, r_idx, t_idx, gamma16, ez, rz)


def _tbody(x_ref, y_ref, o_ref, p_ref):
    t = x_ref[...].T
    o_ref[:, 0:HIDDEN] = t[0:BN // 2]
    o_ref[:, HIDDEN:2 * HIDDEN] = t[BN // 2:BN]
    u = y_ref[...].T
    p_ref[:, 0:HIDDEN] = u[0:BN // 2]
    p_ref[:, HIDDEN:2 * HIDDEN] = u[BN // 2:BN]


@jax.jit
def _relayout(te, tr):
    line = jax.ShapeDtypeStruct((NLINES, 128), jnp.float32)
    return pl.pallas_call(
        _tbody,
        grid=(NB,),
        in_specs=[pl.BlockSpec((HIDDEN, BN), lambda i: (0, i)),
                  pl.BlockSpec((HIDDEN, BN), lambda i: (0, i))],
        out_specs=[pl.BlockSpec((BN // 2, 128), lambda i: (i, 0)),
                   pl.BlockSpec((BN // 2, 128), lambda i: (i, 0))],
        out_shape=[line, line],
    )(te, tr)


def kernel(sample, entity_embedding, relation_embedding, gamma):
    sample = sample.astype(jnp.int32)
    h_idx = sample[:, 0].reshape(NW, BPW)
    r_idx = sample[:, 1].reshape(NW, BPW)
    t_idx = sample[:, 2].reshape(NW, BPW)
    gamma16 = jnp.broadcast_to(gamma.astype(jnp.float32), (16,))
    ez, rz = _relayout(entity_embedding.T, relation_embedding.T)
    scores = _transe_scores(h_idx, r_idx, t_idx, gamma16, ez, rz)
    return scores.reshape(BATCH, 1)
